# Initial kernel scaffold; baseline (speedup 1.0000x reference)
import jax
import jax.numpy as jnp
from jax.experimental import pallas as pl


def kernel(xd1, edge_index1, batch_d1, xd2, edge_index2, batch_d2, xc1, xc2, xc3, xtc, params):
    raise NotImplementedError("write your pallas kernel here")


# speedup vs baseline: 12.9296x; 12.9296x over previous
import functools

import jax
import jax.numpy as jnp
from jax import lax
from jax.experimental import pallas as pl
from jax.experimental.pallas import tpu as pltpu
from jax.experimental.pallas import tpu_sc as plsc

N = 10000
E = 320000
B = 256
D = 128
H1 = 10
NTC = 10
XCL = 256
XD_OUT = 128

NW = 32
NPAD = 10240
EP = 330240
EPT = EP // NW
CE_ATT = 2064
CE_AGG = 688
SUBROWS = NPAD // 16
POOL_ROWS = 264
PAD_DST = 10008

_MESH = plsc.VectorSubcoreMesh(core_axis_name="c", subcore_axis_name="s")
_F32 = jnp.float32


def _mm_h3d_body(x_ref, w_ref, o_ref):
    o_ref[0, 0] = jnp.dot(x_ref[...], w_ref[0, 0], preferred_element_type=_F32)


_mm_h3d = pl.pallas_call(
    _mm_h3d_body,
    grid=(H1, NPAD // 512, 2),
    in_specs=[
        pl.BlockSpec((512, 128), lambda h, r, c: (r, 0)),
        pl.BlockSpec((1, 1, 128, 64), lambda h, r, c: (c, h, 0, 0)),
    ],
    out_specs=pl.BlockSpec((1, 1, 512, 64), lambda h, r, c: (c, h, r, 0)),
    out_shape=jax.ShapeDtypeStruct((2, H1, NPAD, 64), _F32),
)


def _attn_proj_body(x_ref, w_ref, asd_ref, m_ref, scr):
    i = pl.program_id(0)
    y = jnp.dot(x_ref[...], w_ref[...], preferred_element_type=_F32)
    asd_ref[...] = y
    bm = jnp.max(y, axis=0, keepdims=True)

    @pl.when(i == 0)
    def _():
        scr[...] = jnp.broadcast_to(bm, (8, 32))

    @pl.when(i > 0)
    def _():
        scr[0:1] = jnp.maximum(scr[0:1], bm)

    @pl.when(i == pl.num_programs(0) - 1)
    def _():
        v = scr[0:1]
        m = jnp.max(v[:, :16]) + jnp.max(v[:, 16:])
        m_ref[...] = jnp.full((1, 32), m, _F32)


_attn_proj = pl.pallas_call(
    _attn_proj_body,
    grid=(NPAD // 512,),
    in_specs=[
        pl.BlockSpec((512, 128), lambda r: (r, 0)),
        pl.BlockSpec((128, 32), lambda r: (0, 0)),
    ],
    out_specs=[
        pl.BlockSpec((512, 32), lambda r: (r, 0)),
        pl.BlockSpec((1, 32), lambda r: (0, 0)),
    ],
    out_shape=[
        jax.ShapeDtypeStruct((NPAD, 32), _F32),
        jax.ShapeDtypeStruct((1, 32), _F32),
    ],
    scratch_shapes=[pltpu.VMEM((8, 32), _F32)],
)


def _l2_body(acc_ref, dnp_ref, b_ref, w_ref, h2e_ref, m2_ref, mscr):
    rb = pl.program_id(0)
    kb = pl.program_id(1)
    kk = kb // 2
    acc = acc_ref[0, 0]
    dall = dnp_ref[0] + dnp_ref[1]
    sel = lax.broadcasted_iota(jnp.int32, (1, 16), 1) == kk
    d = jnp.sum(jnp.where(sel, dall, 0.0), axis=1)
    d = jnp.where(d > 0.0, d, 1.0)
    z = acc / d[:, None] + b_ref[0]
    z = jnp.where(z > 0.0, z, jnp.exp(jnp.minimum(z, 0.0)) - 1.0)
    contrib = jnp.dot(z, w_ref[0], preferred_element_type=_F32)

    @pl.when(kb == 0)
    def _():
        h2e_ref[...] = contrib

    @pl.when(kb > 0)
    def _():
        h2e_ref[...] += contrib

    @pl.when(kb == pl.num_programs(1) - 1)
    def _():
        blk = h2e_ref[...]
        sm = jnp.max(blk[:, 128])
        dm = jnp.max(blk[:, 129])
        prev_s = jnp.where(rb == 0, -jnp.inf, mscr[0, 0])
        prev_d = jnp.where(rb == 0, -jnp.inf, mscr[0, 1])
        mscr[0, 0] = jnp.maximum(prev_s, sm)
        mscr[0, 1] = jnp.maximum(prev_d, dm)

        @pl.when(rb == pl.num_programs(0) - 1)
        def _():
            m2_ref[...] = jnp.full((1, 32), mscr[0, 0] + mscr[0, 1], _F32)


_l2fused = pl.pallas_call(
    _l2_body,
    grid=(NPAD // 512, 2 * H1),
    in_specs=[
        pl.BlockSpec((1, 1, 512, 64), lambda r, k: (k % 2, k // 2, r, 0)),
        pl.BlockSpec((2, 512, 16), lambda r, k: (0, r, 0)),
        pl.BlockSpec((1, 1, 64), lambda r, k: (k, 0, 0)),
        pl.BlockSpec((1, 64, 256), lambda r, k: (k, 0, 0)),
    ],
    out_specs=[
        pl.BlockSpec((512, 256), lambda r, k: (r, 0)),
        pl.BlockSpec((1, 32), lambda r, k: (0, 0)),
    ],
    out_shape=[
        jax.ShapeDtypeStruct((NPAD, 256), _F32),
        jax.ShapeDtypeStruct((1, 32), _F32),
    ],
    scratch_shapes=[pltpu.SMEM((1, 2), _F32)],
)


def _pool_fcg_body(pp_ref, w_ref, b_ref, o_ref):
    mx = jnp.max(pp_ref[...], axis=0)
    y = jnp.dot(mx, w_ref[...], preferred_element_type=_F32) + b_ref[...]
    o_ref[...] = jnp.maximum(y, 0.0)


_pool_fcg = pl.pallas_call(
    _pool_fcg_body,
    grid=(1,),
    in_specs=[
        pl.BlockSpec((NW, 256, 128), lambda i: (0, 0, 0)),
        pl.BlockSpec((128, 128), lambda i: (0, 0)),
        pl.BlockSpec((1, 128), lambda i: (0, 0)),
    ],
    out_specs=pl.BlockSpec((256, 128), lambda i: (0, 0)),
    out_shape=jax.ShapeDtypeStruct((256, 128), _F32),
)


def _cl1_body(l_ref, r_ref, b_ref, o_ref):
    k = pl.program_id(0)
    y = jnp.dot(l_ref[...], r_ref[...], preferred_element_type=_F32)

    @pl.when(k == 0)
    def _():
        o_ref[...] = y

    @pl.when(k > 0)
    def _():
        o_ref[...] += y

    @pl.when(k == pl.num_programs(0) - 1)
    def _():
        o_ref[...] = jnp.maximum(o_ref[...] + b_ref[...], 0.0)


_cl1 = pl.pallas_call(
    _cl1_body,
    grid=(12,),
    in_specs=[
        pl.BlockSpec((256, 2304), lambda k: (0, k)),
        pl.BlockSpec((2304, 512), lambda k: (k, 0)),
        pl.BlockSpec((1, 512), lambda k: (0, 0)),
    ],
    out_specs=pl.BlockSpec((256, 512), lambda k: (0, 0)),
    out_shape=jax.ShapeDtypeStruct((256, 512), _F32),
)


def _make_mm(kdim, ndim, act):
    def body(l_ref, r_ref, b_ref, o_ref):
        y = jnp.dot(l_ref[...], r_ref[...], preferred_element_type=_F32)
        y = y + b_ref[...]
        if act == "relu":
            y = jnp.maximum(y, 0.0)
        elif act == "clip":
            y = jnp.clip(y, -100.0, 100.0)
        o_ref[...] = y

    return pl.pallas_call(
        body,
        grid=(1,),
        in_specs=[
            pl.BlockSpec((256, kdim), lambda i: (0, 0)),
            pl.BlockSpec((kdim, ndim), lambda i: (0, 0)),
            pl.BlockSpec((1, ndim), lambda i: (0, 0)),
        ],
        out_specs=pl.BlockSpec((256, ndim), lambda i: (0, 0)),
        out_shape=jax.ShapeDtypeStruct((256, ndim), _F32),
    )


_mm_cl2 = _make_mm(512, 256, "none")
_mm_fc1 = _make_mm(640, 2048, "relu")
_mm_fc2 = _make_mm(2048, 256, "relu")
_mm_out = _make_mm(384, 128, "clip")


@functools.partial(
    pl.kernel,
    out_type=(
        jax.ShapeDtypeStruct((EP, 16), _F32),
        jax.ShapeDtypeStruct((2, NPAD, 16), _F32),
    ),
    mesh=_MESH,
    compiler_params=pltpu.CompilerParams(use_tc_tiling_on_sc=False),
    scratch_types=[
        pltpu.VMEM((CE_ATT,), jnp.int32),
        pltpu.VMEM((CE_ATT,), jnp.int32),
        pltpu.VMEM((CE_ATT, 16), _F32),
        pltpu.VMEM((CE_ATT, 16), _F32),
        pltpu.VMEM((CE_ATT, 16), _F32),
        pltpu.VMEM((16,), _F32),
        pltpu.VMEM_SHARED((NPAD, 16), _F32),
        pltpu.SemaphoreType.DMA,
    ],
)
def _sc_att(asrc_hbm, adst_hbm, src_hbm, dst_hbm, mv_hbm, zed_hbm,
            ex_hbm, dnp_hbm, sidx, didx, arows, brows, exbuf, mvec, dsp, sem):
    cid = lax.axis_index("c")
    sid = lax.axis_index("s")
    wid = sid * 2 + cid
    pltpu.sync_copy(zed_hbm.at[pl.ds(sid * SUBROWS, SUBROWS)],
                    dsp.at[pl.ds(sid * SUBROWS, SUBROWS)])
    pltpu.sync_copy(mv_hbm, mvec)
    plsc.subcore_barrier()
    base = wid * EPT

    def chunk(ci, carry):
        off = base + ci * CE_ATT
        pltpu.sync_copy(src_hbm.at[pl.ds(off, CE_ATT)], sidx)
        pltpu.sync_copy(dst_hbm.at[pl.ds(off, CE_ATT)], didx)
        pltpu.async_copy(asrc_hbm.at[sidx], arows, sem).wait()
        pltpu.async_copy(adst_hbm.at[didx], brows, sem).wait()
        mv = mvec[...]

        def inner(j, c2):
            e = arows[j, :] + brows[j, :]
            e = jnp.where(e >= 0.0, e, 0.2 * e) - mv
            exbuf[j, :] = jnp.exp(e)
            return c2

        lax.fori_loop(0, CE_ATT, inner, 0)
        pltpu.sync_copy(exbuf, ex_hbm.at[pl.ds(off, CE_ATT)])
        pltpu.sync_copy(exbuf, dsp.at[didx], add=True)
        return carry

    lax.fori_loop(0, EPT // CE_ATT, chunk, 0)
    plsc.subcore_barrier()
    pltpu.sync_copy(dsp.at[pl.ds(sid * SUBROWS, SUBROWS)],
                    dnp_hbm.at[cid, pl.ds(sid * SUBROWS, SUBROWS)])


def _make_agg(nh):
    @functools.partial(
        pl.kernel,
        out_type=jax.ShapeDtypeStruct((2, nh, NPAD, 64), _F32),
        mesh=_MESH,
        compiler_params=pltpu.CompilerParams(use_tc_tiling_on_sc=False),
        scratch_types=[
            pltpu.VMEM((CE_AGG,), jnp.int32),
            pltpu.VMEM((CE_AGG,), jnp.int32),
            pltpu.VMEM((CE_AGG, 16), _F32),
            pltpu.VMEM((CE_AGG, 64), _F32),
            pltpu.VMEM_SHARED((NPAD, 64), _F32),
            pltpu.SemaphoreType.DMA,
        ],
    )
    def agg(h3d, src_hbm, dst_hbm, ex_hbm, zed_hbm, acc_out,
            sidx, didx, exw, rows, asp, sem):
        cid = lax.axis_index("c")
        sid = lax.axis_index("s")
        ept16 = EP // 16
        base = sid * ept16
        for hh in range(nh):
            pltpu.sync_copy(zed_hbm.at[pl.ds(sid * SUBROWS, SUBROWS)],
                            asp.at[pl.ds(sid * SUBROWS, SUBROWS)])
            plsc.subcore_barrier()

            def chunk(ci, carry):
                off = base + ci * CE_AGG
                pltpu.sync_copy(src_hbm.at[pl.ds(off, CE_AGG)], sidx)
                pltpu.sync_copy(dst_hbm.at[pl.ds(off, CE_AGG)], didx)
                pltpu.sync_copy(ex_hbm.at[pl.ds(off, CE_AGG)], exw)
                pltpu.async_copy(h3d.at[cid].at[hh].at[sidx], rows, sem).wait()

                def inner(j, c2):
                    w = exw[j, :][hh]
                    for k in range(4):
                        sl = pl.ds(k * 16, 16)
                        rows[j, sl] = rows[j, sl] * w
                    return c2

                lax.fori_loop(0, CE_AGG, inner, 0)
                pltpu.sync_copy(rows, asp.at[didx], add=True)
                return carry

            lax.fori_loop(0, ept16 // CE_AGG, chunk, 0)
            plsc.subcore_barrier()
            pltpu.sync_copy(asp.at[pl.ds(sid * SUBROWS, SUBROWS)],
                            acc_out.at[cid, hh, pl.ds(sid * SUBROWS, SUBROWS)])
            plsc.subcore_barrier()

    return agg


_agg10 = _make_agg(H1)
_agg1 = _make_agg(1)


@functools.partial(
    pl.kernel,
    out_type=jax.ShapeDtypeStruct((NW, POOL_ROWS, 128), _F32),
    mesh=_MESH,
    compiler_params=pltpu.CompilerParams(use_tc_tiling_on_sc=False),
    scratch_types=[
        pltpu.VMEM((160, 64), _F32),
        pltpu.VMEM((160, 64), _F32),
        pltpu.VMEM((160, 16), _F32),
        pltpu.VMEM((160, 16), _F32),
        pltpu.VMEM((176,), jnp.int32),
        pltpu.VMEM((128,), _F32),
        pltpu.VMEM((POOL_ROWS, 128), _F32),
    ],
)
def _sc_pool(pp_hbm, dnp_hbm, bias_hbm, batch_hbm, out_hbm,
             p0c, p1c, d0c, d1c, bc, bias, acc):
    cid = lax.axis_index("c")
    sid = lax.axis_index("s")
    wid = sid * 2 + cid
    pltpu.sync_copy(bias_hbm, bias)
    zv = jnp.zeros((16,), _F32)

    def zrow(j, carry):
        for k in range(8):
            acc[j, pl.ds(k * 16, 16)] = zv
        return carry

    lax.fori_loop(0, POOL_ROWS, zrow, 0)
    nb = wid * (NPAD // NW)

    def sub(t, carry):
        off = nb + t * 160
        pltpu.sync_copy(pp_hbm.at[0, pl.ds(off, 160)], p0c)
        pltpu.sync_copy(pp_hbm.at[1, pl.ds(off, 160)], p1c)
        pltpu.sync_copy(dnp_hbm.at[0, pl.ds(off, 160)], d0c)
        pltpu.sync_copy(dnp_hbm.at[1, pl.ds(off, 160)], d1c)
        pltpu.sync_copy(batch_hbm.at[pl.ds(off, 160)], bc.at[pl.ds(0, 160)])

        def inner(j, c2):
            dv = d0c[j, :][0] + d1c[j, :][0]
            dv = jnp.where(dv > 0.0, dv, 1.0)
            bj = bc[pl.ds(j, 16)][0]
            for k in range(8):
                half = p0c if k < 4 else p1c
                hsl = pl.ds((k % 4) * 16, 16)
                sl = pl.ds(k * 16, 16)
                v = half[j, hsl] / dv + bias[sl]
                v = jnp.maximum(v, 0.0)
                acc[bj, sl] = jnp.maximum(acc[bj, sl], v)
            return c2

        lax.fori_loop(0, 160, inner, 0)
        return carry

    lax.fori_loop(0, 2, sub, 0)
    pltpu.sync_copy(acc, out_hbm.at[wid])


def _branch(x, edge_index, batch, p, br):
    W1 = p["W_g1_" + br]
    W1h = W1.reshape(D, H1, 2, 64).transpose(2, 1, 0, 3)
    wa_s = jnp.einsum("dhc,hc->dh", W1.reshape(D, H1, D), p["a_src_g1_" + br])
    wa_d = jnp.einsum("dhc,hc->dh", W1.reshape(D, H1, D), p["a_dst_g1_" + br])
    wattn = jnp.concatenate([
        jnp.pad(wa_s, ((0, 0), (0, 6))), jnp.pad(wa_d, ((0, 0), (0, 6)))], axis=1)

    xp = jnp.pad(x, ((0, NPAD - N), (0, 0)))
    h3d = _mm_h3d(xp, W1h)
    asd, m1vec = _attn_proj(xp, wattn)
    asrc, adst = asd[:, :16], asd[:, 16:]
    m1 = m1vec[0, :16]

    loops = jnp.arange(N, dtype=jnp.int32)
    npad_e = EP - E - N
    src = jnp.concatenate([edge_index[0], loops,
                           jnp.zeros((npad_e,), jnp.int32)])
    dst = jnp.concatenate([edge_index[1], loops,
                           jnp.full((npad_e,), PAD_DST, jnp.int32)])
    zed16 = jnp.zeros((NPAD, 16), _F32)
    zed64 = jnp.zeros((NPAD, 64), _F32)

    ex1, dnp1 = _sc_att(asrc, adst, src, dst, m1, zed16)
    acc4 = _agg10(h3d, src, dst, ex1, zed64)

    W2 = p["W_g2_" + br]
    wa2s = W2 @ p["a_src_g2_" + br][0]
    wa2d = W2 @ p["a_dst_g2_" + br][0]
    w2e = jnp.concatenate([
        W2, wa2s[:, None], wa2d[:, None],
        jnp.zeros((H1 * D, 126), _F32)], axis=1).reshape(2 * H1, 64, 256)
    b1r = p["b_g1_" + br].reshape(2 * H1, 1, 64)

    h2e, m2vec = _l2fused(acc4, dnp1, b1r, w2e)
    asrc2, adst2 = h2e[:, 128:144], h2e[:, 129:145]
    m2 = m2vec[0, :16]

    ex2, dnp2 = _sc_att(asrc2, adst2, src, dst, m2, zed16)
    h2pre = jnp.stack([h2e[:, :64], h2e[:, 64:128]])[:, None]
    acc2 = _agg1(h2pre, src, dst, ex2, zed64)
    pp = acc2.reshape(2, NPAD, 64)

    batchp = jnp.concatenate([batch, jnp.full((NPAD - N,), B, jnp.int32)])
    partials = _sc_pool(pp, dnp2, p["b_g2_" + br], batchp)
    return _pool_fcg(partials, p["W_fcg_" + br],
                     p["b_fcg_" + br].reshape(1, 128))


def kernel(xd1, edge_index1, batch_d1, xd2, edge_index2, batch_d2,
           xc1, xc2, xc3, xtc, params):
    p = params
    h1 = _branch(xd1, edge_index1, batch_d1, p, "d1")
    h2 = _branch(xd2, edge_index2, batch_d2, p, "d2")

    xcl_in = jnp.concatenate(
        [xc1, xc2, xc3, xtc, jnp.zeros((B, 32), _F32)], axis=1)
    w1p = jnp.pad(p["W_cl1"], ((0, 32), (0, 0)))
    hc = _cl1(xcl_in, w1p, p["b_cl1"].reshape(1, 512))
    xcl = _mm_cl2(hc, p["W_cl2"], p["b_cl2"].reshape(1, 256))

    xcin = jnp.concatenate(
        [h1, h2, xcl, xtc, jnp.zeros((B, 118), _F32)], axis=1)
    wf1p = jnp.pad(p["W_fc1"], ((0, 118), (0, 0)))
    y = _mm_fc1(xcin, wf1p, p["b_fc1"].reshape(1, 2048))
    y = _mm_fc2(y, p["W_fc2"], p["b_fc2"].reshape(1, 256))

    xo = jnp.concatenate([y, xtc, jnp.zeros((B, 118), _F32)], axis=1)
    wop = jnp.pad(p["W_out"], ((0, 118), (0, 127)))
    bop = jnp.pad(p["b_out"], (0, 127)).reshape(1, 128)
    out = _mm_out(xo, wop, bop)
    return out[:, :1]


# speedup vs baseline: 16.8822x; 1.3057x over previous
import functools

import jax
import jax.numpy as jnp
from jax import lax
from jax.experimental import pallas as pl
from jax.experimental.pallas import tpu as pltpu
from jax.experimental.pallas import tpu_sc as plsc

N = 10000
E = 320000
B = 256
D = 128
H1 = 10
NTC = 10
XCL = 256
XD_OUT = 128

NW = 32
NPAD = 10240
EP = 330240
EPT = EP // NW
CE_ATT = 2064
CE_AGG = 688
SUBROWS = NPAD // 16
POOL_ROWS = 264
PAD_DST = 10008

_MESH = plsc.VectorSubcoreMesh(core_axis_name="c", subcore_axis_name="s")
_F32 = jnp.float32


def _mm_h3d_body(x_ref, w_ref, o_ref):
    o_ref[0, 0] = jnp.dot(x_ref[...], w_ref[0, 0], preferred_element_type=_F32)


_mm_h3d = pl.pallas_call(
    _mm_h3d_body,
    grid=(H1, NPAD // 512, 2),
    in_specs=[
        pl.BlockSpec((512, 128), lambda h, r, c: (r, 0)),
        pl.BlockSpec((1, 1, 128, 64), lambda h, r, c: (c, h, 0, 0)),
    ],
    out_specs=pl.BlockSpec((1, 1, 512, 64), lambda h, r, c: (c, h, r, 0)),
    out_shape=jax.ShapeDtypeStruct((2, H1, NPAD, 64), _F32),
)


def _attn_proj_body(x_ref, w_ref, asd_ref, m_ref, scr):
    i = pl.program_id(0)
    y = jnp.dot(x_ref[...], w_ref[...], preferred_element_type=_F32)
    asd_ref[...] = y
    bm = jnp.max(y, axis=0, keepdims=True)

    @pl.when(i == 0)
    def _():
        scr[...] = jnp.broadcast_to(bm, (8, 32))

    @pl.when(i > 0)
    def _():
        scr[0:1] = jnp.maximum(scr[0:1], bm)

    @pl.when(i == pl.num_programs(0) - 1)
    def _():
        v = scr[0:1]
        m = jnp.max(v[:, :16]) + jnp.max(v[:, 16:])
        m_ref[...] = jnp.full((1, 32), m, _F32)


_attn_proj = pl.pallas_call(
    _attn_proj_body,
    grid=(NPAD // 512,),
    in_specs=[
        pl.BlockSpec((512, 128), lambda r: (r, 0)),
        pl.BlockSpec((128, 32), lambda r: (0, 0)),
    ],
    out_specs=[
        pl.BlockSpec((512, 32), lambda r: (r, 0)),
        pl.BlockSpec((1, 32), lambda r: (0, 0)),
    ],
    out_shape=[
        jax.ShapeDtypeStruct((NPAD, 32), _F32),
        jax.ShapeDtypeStruct((1, 32), _F32),
    ],
    scratch_shapes=[pltpu.VMEM((8, 32), _F32)],
)


def _l2_body(acc_ref, dnp_ref, b_ref, w_ref, h2e_ref, m2_ref, mscr):
    rb = pl.program_id(0)
    kb = pl.program_id(1)
    kk = kb // 2
    acc = acc_ref[0, 0]
    dall = dnp_ref[0] + dnp_ref[1]
    sel = lax.broadcasted_iota(jnp.int32, (1, 16), 1) == kk
    d = jnp.sum(jnp.where(sel, dall, 0.0), axis=1)
    d = jnp.where(d > 0.0, d, 1.0)
    z = acc / d[:, None] + b_ref[0]
    z = jnp.where(z > 0.0, z, jnp.exp(jnp.minimum(z, 0.0)) - 1.0)
    contrib = jnp.dot(z, w_ref[0], preferred_element_type=_F32)

    @pl.when(kb == 0)
    def _():
        h2e_ref[...] = contrib

    @pl.when(kb > 0)
    def _():
        h2e_ref[...] += contrib

    @pl.when(kb == pl.num_programs(1) - 1)
    def _():
        blk = h2e_ref[...]
        sm = jnp.max(blk[:, 128])
        dm = jnp.max(blk[:, 129])
        prev_s = jnp.where(rb == 0, -jnp.inf, mscr[0, 0])
        prev_d = jnp.where(rb == 0, -jnp.inf, mscr[0, 1])
        mscr[0, 0] = jnp.maximum(prev_s, sm)
        mscr[0, 1] = jnp.maximum(prev_d, dm)

        @pl.when(rb == pl.num_programs(0) - 1)
        def _():
            m2_ref[...] = jnp.full((1, 32), mscr[0, 0] + mscr[0, 1], _F32)


_l2fused = pl.pallas_call(
    _l2_body,
    grid=(NPAD // 512, 2 * H1),
    in_specs=[
        pl.BlockSpec((1, 1, 512, 64), lambda r, k: (k % 2, k // 2, r, 0)),
        pl.BlockSpec((2, 512, 16), lambda r, k: (0, r, 0)),
        pl.BlockSpec((1, 1, 64), lambda r, k: (k, 0, 0)),
        pl.BlockSpec((1, 64, 256), lambda r, k: (k, 0, 0)),
    ],
    out_specs=[
        pl.BlockSpec((512, 256), lambda r, k: (r, 0)),
        pl.BlockSpec((1, 32), lambda r, k: (0, 0)),
    ],
    out_shape=[
        jax.ShapeDtypeStruct((NPAD, 256), _F32),
        jax.ShapeDtypeStruct((1, 32), _F32),
    ],
    scratch_shapes=[pltpu.SMEM((1, 2), _F32)],
)


def _pool_fcg_body(pp_ref, w_ref, b_ref, o_ref):
    mx = jnp.max(pp_ref[...], axis=0)
    y = jnp.dot(mx, w_ref[...], preferred_element_type=_F32) + b_ref[...]
    o_ref[...] = jnp.maximum(y, 0.0)


_pool_fcg = pl.pallas_call(
    _pool_fcg_body,
    grid=(1,),
    in_specs=[
        pl.BlockSpec((NW, 256, 128), lambda i: (0, 0, 0)),
        pl.BlockSpec((128, 128), lambda i: (0, 0)),
        pl.BlockSpec((1, 128), lambda i: (0, 0)),
    ],
    out_specs=pl.BlockSpec((256, 128), lambda i: (0, 0)),
    out_shape=jax.ShapeDtypeStruct((256, 128), _F32),
)


def _cl1_body(l_ref, r_ref, b_ref, o_ref):
    k = pl.program_id(0)
    y = jnp.dot(l_ref[...], r_ref[...], preferred_element_type=_F32)

    @pl.when(k == 0)
    def _():
        o_ref[...] = y

    @pl.when(k > 0)
    def _():
        o_ref[...] += y

    @pl.when(k == pl.num_programs(0) - 1)
    def _():
        o_ref[...] = jnp.maximum(o_ref[...] + b_ref[...], 0.0)


_cl1 = pl.pallas_call(
    _cl1_body,
    grid=(12,),
    in_specs=[
        pl.BlockSpec((256, 2304), lambda k: (0, k)),
        pl.BlockSpec((2304, 512), lambda k: (k, 0)),
        pl.BlockSpec((1, 512), lambda k: (0, 0)),
    ],
    out_specs=pl.BlockSpec((256, 512), lambda k: (0, 0)),
    out_shape=jax.ShapeDtypeStruct((256, 512), _F32),
)


def _make_mm(kdim, ndim, act):
    def body(l_ref, r_ref, b_ref, o_ref):
        y = jnp.dot(l_ref[...], r_ref[...], preferred_element_type=_F32)
        y = y + b_ref[...]
        if act == "relu":
            y = jnp.maximum(y, 0.0)
        elif act == "clip":
            y = jnp.clip(y, -100.0, 100.0)
        o_ref[...] = y

    return pl.pallas_call(
        body,
        grid=(1,),
        in_specs=[
            pl.BlockSpec((256, kdim), lambda i: (0, 0)),
            pl.BlockSpec((kdim, ndim), lambda i: (0, 0)),
            pl.BlockSpec((1, ndim), lambda i: (0, 0)),
        ],
        out_specs=pl.BlockSpec((256, ndim), lambda i: (0, 0)),
        out_shape=jax.ShapeDtypeStruct((256, ndim), _F32),
    )


_mm_cl2 = _make_mm(512, 256, "none")
_mm_fc1 = _make_mm(640, 2048, "relu")
_mm_fc2 = _make_mm(2048, 256, "relu")
_mm_out = _make_mm(384, 128, "clip")


@functools.partial(
    pl.kernel,
    out_type=(
        jax.ShapeDtypeStruct((EP, 16), _F32),
        jax.ShapeDtypeStruct((2, NPAD, 16), _F32),
    ),
    mesh=_MESH,
    compiler_params=pltpu.CompilerParams(use_tc_tiling_on_sc=False),
    scratch_types=[
        pltpu.VMEM((CE_ATT,), jnp.int32),
        pltpu.VMEM((CE_ATT,), jnp.int32),
        pltpu.VMEM((CE_ATT, 16), _F32),
        pltpu.VMEM((CE_ATT, 16), _F32),
        pltpu.VMEM((CE_ATT, 16), _F32),
        pltpu.VMEM((16,), _F32),
        pltpu.VMEM_SHARED((NPAD, 16), _F32),
        pltpu.SemaphoreType.DMA,
    ],
)
def _sc_att(asrc_hbm, adst_hbm, src_hbm, dst_hbm, mv_hbm, zed_hbm,
            ex_hbm, dnp_hbm, sidx, didx, arows, brows, exbuf, mvec, dsp, sem):
    cid = lax.axis_index("c")
    sid = lax.axis_index("s")
    wid = sid * 2 + cid
    pltpu.sync_copy(zed_hbm.at[pl.ds(sid * SUBROWS, SUBROWS)],
                    dsp.at[pl.ds(sid * SUBROWS, SUBROWS)])
    pltpu.sync_copy(mv_hbm, mvec)
    plsc.subcore_barrier()
    base = wid * EPT

    def chunk(ci, carry):
        off = base + ci * CE_ATT
        pltpu.sync_copy(src_hbm.at[pl.ds(off, CE_ATT)], sidx)
        pltpu.sync_copy(dst_hbm.at[pl.ds(off, CE_ATT)], didx)
        d1 = pltpu.async_copy(asrc_hbm.at[sidx], arows, sem)
        d2 = pltpu.async_copy(adst_hbm.at[didx], brows, sem)
        d1.wait()
        d2.wait()
        mv = mvec[...]

        @plsc.parallel_loop(0, CE_ATT, 1, unroll=4)
        def _(j):
            e = arows[j, :] + brows[j, :]
            e = jnp.where(e >= 0.0, e, 0.2 * e) - mv
            exbuf[j, :] = jnp.exp(e)
        pltpu.sync_copy(exbuf, ex_hbm.at[pl.ds(off, CE_ATT)])
        pltpu.sync_copy(exbuf, dsp.at[didx], add=True)
        return carry

    lax.fori_loop(0, EPT // CE_ATT, chunk, 0)
    plsc.subcore_barrier()
    pltpu.sync_copy(dsp.at[pl.ds(sid * SUBROWS, SUBROWS)],
                    dnp_hbm.at[cid, pl.ds(sid * SUBROWS, SUBROWS)])


def _make_agg(nh):
    @functools.partial(
        pl.kernel,
        out_type=jax.ShapeDtypeStruct((2, nh, NPAD, 64), _F32),
        mesh=_MESH,
        compiler_params=pltpu.CompilerParams(use_tc_tiling_on_sc=False),
        scratch_types=[
            pltpu.VMEM((CE_AGG,), jnp.int32),
            pltpu.VMEM((CE_AGG,), jnp.int32),
            pltpu.VMEM((CE_AGG, 16), _F32),
            pltpu.VMEM((CE_AGG, 64), _F32),
            pltpu.VMEM_SHARED((NPAD, 64), _F32),
            pltpu.SemaphoreType.DMA,
        ],
    )
    def agg(h3d, src_hbm, dst_hbm, ex_hbm, zed_hbm, acc_out,
            sidx, didx, exw, rows, asp, sem):
        cid = lax.axis_index("c")
        sid = lax.axis_index("s")
        ept16 = EP // 16
        base = sid * ept16
        for hh in range(nh):
            pltpu.sync_copy(zed_hbm.at[pl.ds(sid * SUBROWS, SUBROWS)],
                            asp.at[pl.ds(sid * SUBROWS, SUBROWS)])
            plsc.subcore_barrier()

            def chunk(ci, carry):
                off = base + ci * CE_AGG
                pltpu.sync_copy(src_hbm.at[pl.ds(off, CE_AGG)], sidx)
                pltpu.sync_copy(dst_hbm.at[pl.ds(off, CE_AGG)], didx)
                pltpu.sync_copy(ex_hbm.at[pl.ds(off, CE_AGG)], exw)
                pltpu.async_copy(h3d.at[cid].at[hh].at[sidx], rows, sem).wait()

                @plsc.parallel_loop(0, CE_AGG, 1, unroll=4)
                def _(j):
                    w = exw[j, :][hh]
                    for k in range(4):
                        sl = pl.ds(k * 16, 16)
                        rows[j, sl] = rows[j, sl] * w
                pltpu.sync_copy(rows, asp.at[didx], add=True)
                return carry

            lax.fori_loop(0, ept16 // CE_AGG, chunk, 0)
            plsc.subcore_barrier()
            pltpu.sync_copy(asp.at[pl.ds(sid * SUBROWS, SUBROWS)],
                            acc_out.at[cid, hh, pl.ds(sid * SUBROWS, SUBROWS)])
            plsc.subcore_barrier()

    return agg


_agg10 = _make_agg(H1)
_agg1 = _make_agg(1)


@functools.partial(
    pl.kernel,
    out_type=jax.ShapeDtypeStruct((NW, POOL_ROWS, 128), _F32),
    mesh=_MESH,
    compiler_params=pltpu.CompilerParams(use_tc_tiling_on_sc=False),
    scratch_types=[
        pltpu.VMEM((160, 64), _F32),
        pltpu.VMEM((160, 64), _F32),
        pltpu.VMEM((160, 16), _F32),
        pltpu.VMEM((160, 16), _F32),
        pltpu.VMEM((176,), jnp.int32),
        pltpu.VMEM((128,), _F32),
        pltpu.VMEM((POOL_ROWS, 128), _F32),
    ],
)
def _sc_pool(pp_hbm, dnp_hbm, bias_hbm, batch_hbm, out_hbm,
             p0c, p1c, d0c, d1c, bc, bias, acc):
    cid = lax.axis_index("c")
    sid = lax.axis_index("s")
    wid = sid * 2 + cid
    pltpu.sync_copy(bias_hbm, bias)
    zv = jnp.zeros((16,), _F32)

    def zrow(j, carry):
        for k in range(8):
            acc[j, pl.ds(k * 16, 16)] = zv
        return carry

    lax.fori_loop(0, POOL_ROWS, zrow, 0)
    nb = wid * (NPAD // NW)

    def sub(t, carry):
        off = nb + t * 160
        pltpu.sync_copy(pp_hbm.at[0, pl.ds(off, 160)], p0c)
        pltpu.sync_copy(pp_hbm.at[1, pl.ds(off, 160)], p1c)
        pltpu.sync_copy(dnp_hbm.at[0, pl.ds(off, 160)], d0c)
        pltpu.sync_copy(dnp_hbm.at[1, pl.ds(off, 160)], d1c)
        pltpu.sync_copy(batch_hbm.at[pl.ds(off, 160)], bc.at[pl.ds(0, 160)])

        def inner(j, c2):
            dv = d0c[j, :][0] + d1c[j, :][0]
            dv = jnp.where(dv > 0.0, dv, 1.0)
            bj = bc[pl.ds(j, 16)][0]
            for k in range(8):
                half = p0c if k < 4 else p1c
                hsl = pl.ds((k % 4) * 16, 16)
                sl = pl.ds(k * 16, 16)
                v = half[j, hsl] / dv + bias[sl]
                v = jnp.maximum(v, 0.0)
                acc[bj, sl] = jnp.maximum(acc[bj, sl], v)
            return c2

        lax.fori_loop(0, 160, inner, 0)
        return carry

    lax.fori_loop(0, 2, sub, 0)
    pltpu.sync_copy(acc, out_hbm.at[wid])


def _branch(x, edge_index, batch, p, br):
    W1 = p["W_g1_" + br]
    W1h = W1.reshape(D, H1, 2, 64).transpose(2, 1, 0, 3)
    wa_s = jnp.einsum("dhc,hc->dh", W1.reshape(D, H1, D), p["a_src_g1_" + br])
    wa_d = jnp.einsum("dhc,hc->dh", W1.reshape(D, H1, D), p["a_dst_g1_" + br])
    wattn = jnp.concatenate([
        jnp.pad(wa_s, ((0, 0), (0, 6))), jnp.pad(wa_d, ((0, 0), (0, 6)))], axis=1)

    xp = jnp.pad(x, ((0, NPAD - N), (0, 0)))
    h3d = _mm_h3d(xp, W1h)
    asd, m1vec = _attn_proj(xp, wattn)
    asrc, adst = asd[:, :16], asd[:, 16:]
    m1 = m1vec[0, :16]

    loops = jnp.arange(N, dtype=jnp.int32)
    npad_e = EP - E - N
    src = jnp.concatenate([edge_index[0], loops,
                           jnp.zeros((npad_e,), jnp.int32)])
    dst = jnp.concatenate([edge_index[1], loops,
                           jnp.full((npad_e,), PAD_DST, jnp.int32)])
    zed16 = jnp.zeros((NPAD, 16), _F32)
    zed64 = jnp.zeros((NPAD, 64), _F32)

    ex1, dnp1 = _sc_att(asrc, adst, src, dst, m1, zed16)
    acc4 = _agg10(h3d, src, dst, ex1, zed64)

    W2 = p["W_g2_" + br]
    wa2s = W2 @ p["a_src_g2_" + br][0]
    wa2d = W2 @ p["a_dst_g2_" + br][0]
    w2e = jnp.concatenate([
        W2, wa2s[:, None], wa2d[:, None],
        jnp.zeros((H1 * D, 126), _F32)], axis=1).reshape(2 * H1, 64, 256)
    b1r = p["b_g1_" + br].reshape(2 * H1, 1, 64)

    h2e, m2vec = _l2fused(acc4, dnp1, b1r, w2e)
    asrc2, adst2 = h2e[:, 128:144], h2e[:, 129:145]
    m2 = m2vec[0, :16]

    ex2, dnp2 = _sc_att(asrc2, adst2, src, dst, m2, zed16)
    h2pre = jnp.stack([h2e[:, :64], h2e[:, 64:128]])[:, None]
    acc2 = _agg1(h2pre, src, dst, ex2, zed64)
    pp = acc2.reshape(2, NPAD, 64)

    batchp = jnp.concatenate([batch, jnp.full((NPAD - N,), B, jnp.int32)])
    partials = _sc_pool(pp, dnp2, p["b_g2_" + br], batchp)
    return _pool_fcg(partials, p["W_fcg_" + br],
                     p["b_fcg_" + br].reshape(1, 128))


def kernel(xd1, edge_index1, batch_d1, xd2, edge_index2, batch_d2,
           xc1, xc2, xc3, xtc, params):
    p = params
    h1 = _branch(xd1, edge_index1, batch_d1, p, "d1")
    h2 = _branch(xd2, edge_index2, batch_d2, p, "d2")

    xcl_in = jnp.concatenate(
        [xc1, xc2, xc3, xtc, jnp.zeros((B, 32), _F32)], axis=1)
    w1p = jnp.pad(p["W_cl1"], ((0, 32), (0, 0)))
    hc = _cl1(xcl_in, w1p, p["b_cl1"].reshape(1, 512))
    xcl = _mm_cl2(hc, p["W_cl2"], p["b_cl2"].reshape(1, 256))

    xcin = jnp.concatenate(
        [h1, h2, xcl, xtc, jnp.zeros((B, 118), _F32)], axis=1)
    wf1p = jnp.pad(p["W_fc1"], ((0, 118), (0, 0)))
    y = _mm_fc1(xcin, wf1p, p["b_fc1"].reshape(1, 2048))
    y = _mm_fc2(y, p["W_fc2"], p["b_fc2"].reshape(1, 256))

    xo = jnp.concatenate([y, xtc, jnp.zeros((B, 118), _F32)], axis=1)
    wop = jnp.pad(p["W_out"], ((0, 118), (0, 127)))
    bop = jnp.pad(p["b_out"], (0, 127)).reshape(1, 128)
    out = _mm_out(xo, wop, bop)
    return out[:, :1]
